# baseline (speedup 1.0000x reference)
import jax
import jax.numpy as jnp
from jax.experimental import pallas as pl

B, MAX_ATOMS, MAX_DEGREE = 512, 100, 6
NUM_ATOM_FEATURES, NUM_BOND_FEATURES, CONV_WIDTH = 128, 16, 128
BM = 8


def _body(atoms_ref, bonds_ref, edges_ref, W_ref, b_ref, out_ref):
    e = edges_ref[...]
    s_bond = jnp.sum(bonds_ref[...], axis=2)

    lane = jax.lax.broadcasted_iota(jnp.int32, (MAX_ATOMS, MAX_ATOMS), 1)
    for m in range(BM):
        e_m = e[m]
        at_m = atoms_ref[m]
        amat = jnp.zeros((MAX_ATOMS, MAX_ATOMS), dtype=jnp.float32)
        for d in range(MAX_DEGREE):
            amat = amat + jnp.where(e_m[:, d:d + 1] == lane, 1.0, 0.0)
        s_atom_m = at_m + jax.lax.dot(amat, at_m, preferred_element_type=jnp.float32)
        s_bond_m = s_bond[m]

        slot_valid = [e_m[:, d:d + 1] >= 0 for d in range(MAX_DEGREE)]
        acc = jnp.zeros((MAX_ATOMS, CONV_WIDTH), dtype=jnp.float32)
        for d in range(MAX_DEGREE):
            y = (
                jax.lax.dot(s_atom_m, W_ref[d, :NUM_ATOM_FEATURES, :],
                            preferred_element_type=jnp.float32)
                + jax.lax.dot(s_bond_m, W_ref[d, NUM_ATOM_FEATURES:, :],
                              preferred_element_type=jnp.float32)
                + b_ref[d][None, :]
            )
            y = jax.nn.relu(y)
            if d == 0:
                mask = ~slot_valid[0]
            else:
                mask = slot_valid[d - 1] & ~slot_valid[d] if d < MAX_DEGREE else slot_valid[d - 1]
            acc = acc + jnp.where(mask, y, 0.0)
        out_ref[m] = acc


@jax.jit
def kernel(atoms, bonds, edges, W, b):
    return pl.pallas_call(
        _body,
        grid=(B // BM,),
        in_specs=[
            pl.BlockSpec((BM, MAX_ATOMS, NUM_ATOM_FEATURES), lambda i: (i, 0, 0)),
            pl.BlockSpec((BM, MAX_ATOMS, MAX_DEGREE, NUM_BOND_FEATURES),
                         lambda i: (i, 0, 0, 0)),
            pl.BlockSpec((BM, MAX_ATOMS, MAX_DEGREE), lambda i: (i, 0, 0)),
            pl.BlockSpec((MAX_DEGREE, NUM_ATOM_FEATURES + NUM_BOND_FEATURES, CONV_WIDTH),
                         lambda i: (0, 0, 0)),
            pl.BlockSpec((MAX_DEGREE, CONV_WIDTH), lambda i: (0, 0)),
        ],
        out_specs=pl.BlockSpec((BM, MAX_ATOMS, CONV_WIDTH), lambda i: (i, 0, 0)),
        out_shape=jax.ShapeDtypeStruct((B, MAX_ATOMS, CONV_WIDTH), jnp.float32),
    )(atoms, bonds, edges.astype(jnp.int32), W, b)


# speedup vs baseline: 1.2224x; 1.2224x over previous
import jax
import jax.numpy as jnp
from jax.experimental import pallas as pl

B, MAX_ATOMS, MAX_DEGREE = 512, 100, 6
NUM_ATOM_FEATURES, NUM_BOND_FEATURES, CONV_WIDTH = 128, 16, 128
BM = 8


def _body(atoms_ref, bonds_ref, edges_ref, W_ref, b_ref, out_ref):
    e = edges_ref[...]

    bsel_i = jax.lax.broadcasted_iota(jnp.int32, (MAX_DEGREE * NUM_BOND_FEATURES, NUM_BOND_FEATURES), 0)
    bsel_j = jax.lax.broadcasted_iota(jnp.int32, (MAX_DEGREE * NUM_BOND_FEATURES, NUM_BOND_FEATURES), 1)
    bsel = jnp.where(bsel_i % NUM_BOND_FEATURES == bsel_j, 1.0, 0.0)
    s_bond = jax.lax.dot(bonds_ref[...], bsel, preferred_element_type=jnp.float32)

    lane = jax.lax.broadcasted_iota(jnp.int32, (MAX_ATOMS, MAX_ATOMS), 1)
    for m in range(BM):
        e_m = e[m]
        at_m = atoms_ref[m]
        amat = jnp.zeros((MAX_ATOMS, MAX_ATOMS), dtype=jnp.float32)
        for d in range(MAX_DEGREE):
            amat = amat + jnp.where(e_m[:, d:d + 1] == lane, 1.0, 0.0)
        s_atom_m = at_m + jax.lax.dot(amat, at_m, preferred_element_type=jnp.float32)
        s_bond_m = s_bond[m * MAX_ATOMS:(m + 1) * MAX_ATOMS, :]

        slot_valid = [e_m[:, d:d + 1] >= 0 for d in range(MAX_DEGREE)]
        acc = jnp.zeros((MAX_ATOMS, CONV_WIDTH), dtype=jnp.float32)
        for d in range(MAX_DEGREE):
            y = (
                jax.lax.dot(s_atom_m, W_ref[d, :NUM_ATOM_FEATURES, :],
                            preferred_element_type=jnp.float32)
                + jax.lax.dot(s_bond_m, W_ref[d, NUM_ATOM_FEATURES:, :],
                              preferred_element_type=jnp.float32)
                + b_ref[d][None, :]
            )
            y = jax.nn.relu(y)
            if d == 0:
                mask = ~slot_valid[0]
            else:
                mask = slot_valid[d - 1] & ~slot_valid[d]
            acc = acc + jnp.where(mask, y, 0.0)
        out_ref[m] = acc


@jax.jit
def kernel(atoms, bonds, edges, W, b):
    bonds2d = bonds.reshape(B * MAX_ATOMS, MAX_DEGREE * NUM_BOND_FEATURES)
    return pl.pallas_call(
        _body,
        grid=(B // BM,),
        in_specs=[
            pl.BlockSpec((BM, MAX_ATOMS, NUM_ATOM_FEATURES), lambda i: (i, 0, 0)),
            pl.BlockSpec((BM * MAX_ATOMS, MAX_DEGREE * NUM_BOND_FEATURES), lambda i: (i, 0)),
            pl.BlockSpec((BM, MAX_ATOMS, MAX_DEGREE), lambda i: (i, 0, 0)),
            pl.BlockSpec((MAX_DEGREE, NUM_ATOM_FEATURES + NUM_BOND_FEATURES, CONV_WIDTH),
                         lambda i: (0, 0, 0)),
            pl.BlockSpec((MAX_DEGREE, CONV_WIDTH), lambda i: (0, 0)),
        ],
        out_specs=pl.BlockSpec((BM, MAX_ATOMS, CONV_WIDTH), lambda i: (i, 0, 0)),
        out_shape=jax.ShapeDtypeStruct((B, MAX_ATOMS, CONV_WIDTH), jnp.float32),
    )(atoms, bonds2d, edges.astype(jnp.int32), W, b)
